# Initial kernel scaffold; baseline (speedup 1.0000x reference)
import jax
import jax.numpy as jnp
from jax.experimental import pallas as pl


def kernel(x, edge_index, W1, b1, W2, b2):
    raise NotImplementedError("write your pallas kernel here")


# speedup vs baseline: 42.3035x; 42.3035x over previous
import functools

import jax
import jax.numpy as jnp
from jax import lax
from jax.experimental import pallas as pl
from jax.experimental.pallas import tpu as pltpu
from jax.experimental.pallas import tpu_sc as plsc

N_NODES = 10000
N_PAD = 10240
N_EDGES = 320000
D_FEAT = 128
HIDDEN = 64

NC = 2
NS = 16
NW = NC * NS
EW = N_EDGES // NW
C = 100
NCH = EW // C
ZR = N_PAD // NS
NBUF = 4


def _mesh():
    return plsc.VectorSubcoreMesh(core_axis_name="c", subcore_axis_name="s")


_SC_PARAMS = pltpu.CompilerParams(use_tc_tiling_on_sc=False)


def _sc_degree(dst3, ones_h, zeros_h):
    @functools.partial(
        pl.kernel,
        mesh=_mesh(),
        out_type=jax.ShapeDtypeStruct((NC, N_PAD, 1), jnp.float32),
        scratch_types=[
            pltpu.VMEM((NCH, C), jnp.int32),
            pltpu.VMEM((C, 1), jnp.float32),
            pltpu.VMEM_SHARED((N_PAD, 1), jnp.float32),
        ],
        compiler_params=_SC_PARAMS,
    )
    def k(dst_h, ones_hbm, zeros_hbm, out_h, dst_v, ones_v, acc):
        c = lax.axis_index("c")
        s = lax.axis_index("s")
        wid = s * NC + c
        pltpu.sync_copy(dst_h.at[wid], dst_v)
        pltpu.sync_copy(ones_hbm, ones_v)
        pltpu.sync_copy(zeros_hbm.at[pl.ds(s * ZR, ZR)], acc.at[pl.ds(s * ZR, ZR)])
        plsc.subcore_barrier()

        def body(j, _):
            pltpu.sync_copy(ones_v, acc.at[dst_v.at[j]], add=True)
            return _

        lax.fori_loop(0, NCH, body, None)
        plsc.subcore_barrier()
        pltpu.sync_copy(acc.at[pl.ds(s * ZR, ZR)], out_h.at[c, pl.ds(s * ZR, ZR)])

    return k(dst3, ones_h, zeros_h)


def _sc_gather_scatter(table, src3, dst3, zeros_h, feat):
    @functools.partial(
        pl.kernel,
        mesh=_mesh(),
        out_type=jax.ShapeDtypeStruct((NC, N_PAD, feat), jnp.float32),
        scratch_types=[
            pltpu.VMEM((NCH, C), jnp.int32),
            pltpu.VMEM((NCH, C), jnp.int32),
            pltpu.VMEM((NBUF, C, feat), jnp.float32),
            pltpu.VMEM_SHARED((N_PAD, feat), jnp.float32),
            pltpu.SemaphoreType.DMA,
            pltpu.SemaphoreType.DMA,
            pltpu.SemaphoreType.DMA,
            pltpu.SemaphoreType.DMA,
        ],
        compiler_params=_SC_PARAMS,
    )
    def k(tab_h, src_h, dst_h, zeros_hbm, out_h, src_v, dst_v, rows_v, acc,
          sem0, sem1, sem2, sem3):
        sems = (sem0, sem1, sem2, sem3)
        c = lax.axis_index("c")
        s = lax.axis_index("s")
        wid = s * NC + c
        pltpu.sync_copy(src_h.at[wid], src_v)
        pltpu.sync_copy(dst_h.at[wid], dst_v)
        for b in range(NBUF):
            pltpu.async_copy(tab_h.at[src_v.at[b]], rows_v.at[b], sems[b])
        pltpu.sync_copy(zeros_hbm.at[pl.ds(s * ZR, ZR)], acc.at[pl.ds(s * ZR, ZR)])
        plsc.subcore_barrier()

        def body(t, _):
            for b in range(NBUF):
                j = t * NBUF + b
                pltpu.make_async_copy(tab_h.at[src_v.at[j]], rows_v.at[b],
                                      sems[b]).wait()
                pltpu.sync_copy(rows_v.at[b], acc.at[dst_v.at[j]], add=True)
                pltpu.async_copy(tab_h.at[src_v.at[j + NBUF]], rows_v.at[b],
                                 sems[b])
            return _

        lax.fori_loop(0, NCH // NBUF - 1, body, None)
        for b in range(NBUF):
            j = NCH - NBUF + b
            pltpu.make_async_copy(tab_h.at[src_v.at[j]], rows_v.at[b],
                                  sems[b]).wait()
            pltpu.sync_copy(rows_v.at[b], acc.at[dst_v.at[j]], add=True)
        plsc.subcore_barrier()
        pltpu.sync_copy(acc.at[pl.ds(s * ZR, ZR)], out_h.at[c, pl.ds(s * ZR, ZR)])

    return k(table, src3, dst3, zeros_h)


_RB = 2048


def _tc1(x_p, W1, d0, d1):
    def body(x_ref, w_ref, d0_ref, d1_ref, hp_ref, isd_ref):
        deg = d0_ref[...] + d1_ref[...] + 1.0
        isd = lax.rsqrt(deg)
        h = jnp.dot(x_ref[...], w_ref[...], preferred_element_type=jnp.float32)
        hp_ref[...] = h * isd
        isd_ref[...] = isd

    grid = (N_PAD // _RB,)
    return pl.pallas_call(
        body,
        grid=grid,
        in_specs=[
            pl.BlockSpec((_RB, D_FEAT), lambda i: (i, 0)),
            pl.BlockSpec((D_FEAT, HIDDEN), lambda i: (0, 0)),
            pl.BlockSpec((_RB, 1), lambda i: (i, 0)),
            pl.BlockSpec((_RB, 1), lambda i: (i, 0)),
        ],
        out_specs=[
            pl.BlockSpec((_RB, HIDDEN), lambda i: (i, 0)),
            pl.BlockSpec((_RB, 1), lambda i: (i, 0)),
        ],
        out_shape=[
            jax.ShapeDtypeStruct((N_PAD, HIDDEN), jnp.float32),
            jax.ShapeDtypeStruct((N_PAD, 1), jnp.float32),
        ],
    )(x_p, W1, d0, d1)


def _tc2(p0, p1, hp, isd, w2row, b1row):
    def body(p0_ref, p1_ref, hp_ref, isd_ref, w2_ref, b1_ref, sp_ref):
        isd = isd_ref[...]
        out1 = isd * (p0_ref[...] + p1_ref[...] + hp_ref[...]) + b1_ref[...]
        r = jnp.maximum(out1, 0.0)
        sval = jnp.sum(r * w2_ref[...], axis=1, keepdims=True)
        sp_ref[...] = sval * isd

    grid = (N_PAD // _RB,)
    return pl.pallas_call(
        body,
        grid=grid,
        in_specs=[
            pl.BlockSpec((_RB, HIDDEN), lambda i: (i, 0)),
            pl.BlockSpec((_RB, HIDDEN), lambda i: (i, 0)),
            pl.BlockSpec((_RB, HIDDEN), lambda i: (i, 0)),
            pl.BlockSpec((_RB, 1), lambda i: (i, 0)),
            pl.BlockSpec((1, HIDDEN), lambda i: (0, 0)),
            pl.BlockSpec((1, HIDDEN), lambda i: (0, 0)),
        ],
        out_specs=pl.BlockSpec((_RB, 1), lambda i: (i, 0)),
        out_shape=jax.ShapeDtypeStruct((N_PAD, 1), jnp.float32),
    )(p0, p1, hp, isd, w2row, b1row)


def _tc3(q0, q1, sp, isd, b2):
    def body(q0_ref, q1_ref, sp_ref, isd_ref, b2_ref, out_ref):
        out_ref[...] = (isd_ref[...] * (q0_ref[...] + q1_ref[...] + sp_ref[...])
                        + b2_ref[0, 0])

    return pl.pallas_call(
        body,
        in_specs=[
            pl.BlockSpec((N_PAD, 1), lambda: (0, 0)),
            pl.BlockSpec((N_PAD, 1), lambda: (0, 0)),
            pl.BlockSpec((N_PAD, 1), lambda: (0, 0)),
            pl.BlockSpec((N_PAD, 1), lambda: (0, 0)),
            pl.BlockSpec((1, 1), lambda: (0, 0)),
        ],
        out_specs=pl.BlockSpec((N_PAD, 1), lambda: (0, 0)),
        out_shape=jax.ShapeDtypeStruct((N_PAD, 1), jnp.float32),
    )(q0, q1, sp, isd, b2)


def kernel(x, edge_index, W1, b1, W2, b2):
    src3 = edge_index[0].reshape(NW, NCH, C)
    dst3 = edge_index[1].reshape(NW, NCH, C)
    x_p = jnp.pad(x, ((0, N_PAD - N_NODES), (0, 0)))
    ones_h = jnp.ones((C, 1), jnp.float32)
    zeros1 = jnp.zeros((N_PAD, 1), jnp.float32)
    zeros64 = jnp.zeros((N_PAD, HIDDEN), jnp.float32)

    dp = _sc_degree(dst3, ones_h, zeros1)
    hp, isd = _tc1(x_p, W1, dp[0], dp[1])
    pp = _sc_gather_scatter(hp, src3, dst3, zeros64, HIDDEN)
    sp = _tc2(pp[0], pp[1], hp, isd, W2.reshape(1, HIDDEN), b1.reshape(1, HIDDEN))
    qp = _sc_gather_scatter(sp, src3, dst3, zeros1, 1)
    out = _tc3(qp[0], qp[1], sp, isd, b2.reshape(1, 1))
    return out[:N_NODES, 0]
